# baseline (speedup 1.0000x reference)
import jax
import jax.numpy as jnp
from jax.experimental import pallas as pl
from jax.experimental.pallas import tpu as pltpu


def _layer1_body(a_ref, x1_ref, x2_ref, wz_ref, wx_ref, z1_ref, z2_ref, s2_ref):
    f = wz_ref.shape[1]
    t1 = jnp.dot(a_ref[...], x1_ref[...], preferred_element_type=jnp.float32)
    t2 = jnp.dot(a_ref[...], x2_ref[...], preferred_element_type=jnp.float32)
    z1 = jnp.dot(t1, wz_ref[...], preferred_element_type=jnp.float32)
    z2 = jnp.dot(t2, wz_ref[...], preferred_element_type=jnp.float32)
    z1_ref[...] = z1
    z2_ref[...] = z2
    s2_ref[:, :f] = jnp.dot(z1, wx_ref[...], preferred_element_type=jnp.float32)
    s2_ref[:, f:] = jnp.dot(z2, wx_ref[...], preferred_element_type=jnp.float32)


def _layer2_body(a_ref, s_ref, o1_ref, o2_ref):
    f = o1_ref.shape[1]
    out = jnp.dot(a_ref[...], s_ref[...], preferred_element_type=jnp.float32)
    o1_ref[...] = out[:, :f]
    o2_ref[...] = out[:, f:]


def _pick_block(n, target):
    for bm in range(min(target, n), 7, -1):
        if n % bm == 0 and bm % 8 == 0:
            return bm
    return n


def kernel(feat, feat_a, fadj, W_z, W_x):
    n, fin = feat.shape
    fhid = W_z.shape[1]
    fout = W_x.shape[1]
    bm = _pick_block(n, 200)
    grid = (n // bm,)

    z_ori, z_aug, s2 = pl.pallas_call(
        _layer1_body,
        grid=grid,
        in_specs=[
            pl.BlockSpec((bm, n), lambda i: (i, 0)),
            pl.BlockSpec((n, fin), lambda i: (0, 0)),
            pl.BlockSpec((n, fin), lambda i: (0, 0)),
            pl.BlockSpec((fin, fhid), lambda i: (0, 0)),
            pl.BlockSpec((fhid, fout), lambda i: (0, 0)),
        ],
        out_specs=[
            pl.BlockSpec((bm, fhid), lambda i: (i, 0)),
            pl.BlockSpec((bm, fhid), lambda i: (i, 0)),
            pl.BlockSpec((bm, 2 * fout), lambda i: (i, 0)),
        ],
        out_shape=[
            jax.ShapeDtypeStruct((n, fhid), jnp.float32),
            jax.ShapeDtypeStruct((n, fhid), jnp.float32),
            jax.ShapeDtypeStruct((n, 2 * fout), jnp.float32),
        ],
        compiler_params=pltpu.CompilerParams(
            dimension_semantics=("parallel",)),
    )(fadj, feat, feat_a, W_z, W_x)

    xhat_ori, xhat_aug = pl.pallas_call(
        _layer2_body,
        grid=grid,
        in_specs=[
            pl.BlockSpec((bm, n), lambda i: (i, 0)),
            pl.BlockSpec((n, 2 * fout), lambda i: (0, 0)),
        ],
        out_specs=[
            pl.BlockSpec((bm, fout), lambda i: (i, 0)),
            pl.BlockSpec((bm, fout), lambda i: (i, 0)),
        ],
        out_shape=[
            jax.ShapeDtypeStruct((n, fout), jnp.float32),
            jax.ShapeDtypeStruct((n, fout), jnp.float32),
        ],
        compiler_params=pltpu.CompilerParams(
            dimension_semantics=("parallel",)),
    )(fadj, s2)

    return (z_ori, z_aug, xhat_ori, xhat_aug)


# speedup vs baseline: 1.0729x; 1.0729x over previous
import jax
import jax.numpy as jnp
from jax.experimental import pallas as pl
from jax.experimental.pallas import tpu as pltpu


def _layer1_body(a_ref, x1_ref, x2_ref, wz_ref, wx_ref, z1_ref, z2_ref, s2_ref):
    f = wz_ref.shape[1]
    t1 = jnp.dot(a_ref[...], x1_ref[...], preferred_element_type=jnp.float32)
    t2 = jnp.dot(a_ref[...], x2_ref[...], preferred_element_type=jnp.float32)
    z1 = jnp.dot(t1, wz_ref[...], preferred_element_type=jnp.float32)
    z2 = jnp.dot(t2, wz_ref[...], preferred_element_type=jnp.float32)
    z1_ref[...] = z1
    z2_ref[...] = z2
    s2_ref[:, :f] = jnp.dot(z1, wx_ref[...], preferred_element_type=jnp.float32)
    s2_ref[:, f:] = jnp.dot(z2, wx_ref[...], preferred_element_type=jnp.float32)


def _layer2_body(a_ref, s_ref, o1_ref, o2_ref):
    f = o1_ref.shape[1]
    out = jnp.dot(a_ref[...], s_ref[...], preferred_element_type=jnp.float32)
    o1_ref[...] = out[:, :f]
    o2_ref[...] = out[:, f:]


def _pick_block(n, target):
    for bm in range(min(target, n), 7, -1):
        if n % bm == 0 and bm % 8 == 0:
            return bm
    return n


def kernel(feat, feat_a, fadj, W_z, W_x):
    n, fin = feat.shape
    fhid = W_z.shape[1]
    fout = W_x.shape[1]
    bm = _pick_block(n, 500)
    grid = (n // bm,)

    z_ori, z_aug, s2 = pl.pallas_call(
        _layer1_body,
        grid=grid,
        in_specs=[
            pl.BlockSpec((bm, n), lambda i: (i, 0)),
            pl.BlockSpec((n, fin), lambda i: (0, 0)),
            pl.BlockSpec((n, fin), lambda i: (0, 0)),
            pl.BlockSpec((fin, fhid), lambda i: (0, 0)),
            pl.BlockSpec((fhid, fout), lambda i: (0, 0)),
        ],
        out_specs=[
            pl.BlockSpec((bm, fhid), lambda i: (i, 0)),
            pl.BlockSpec((bm, fhid), lambda i: (i, 0)),
            pl.BlockSpec((bm, 2 * fout), lambda i: (i, 0)),
        ],
        out_shape=[
            jax.ShapeDtypeStruct((n, fhid), jnp.float32),
            jax.ShapeDtypeStruct((n, fhid), jnp.float32),
            jax.ShapeDtypeStruct((n, 2 * fout), jnp.float32),
        ],
        compiler_params=pltpu.CompilerParams(
            dimension_semantics=("parallel",)),
    )(fadj, feat, feat_a, W_z, W_x)

    xhat_ori, xhat_aug = pl.pallas_call(
        _layer2_body,
        grid=grid,
        in_specs=[
            pl.BlockSpec((bm, n), lambda i: (i, 0)),
            pl.BlockSpec((n, 2 * fout), lambda i: (0, 0)),
        ],
        out_specs=[
            pl.BlockSpec((bm, fout), lambda i: (i, 0)),
            pl.BlockSpec((bm, fout), lambda i: (i, 0)),
        ],
        out_shape=[
            jax.ShapeDtypeStruct((n, fout), jnp.float32),
            jax.ShapeDtypeStruct((n, fout), jnp.float32),
        ],
        compiler_params=pltpu.CompilerParams(
            dimension_semantics=("parallel",)),
    )(fadj, s2)

    return (z_ori, z_aug, xhat_ori, xhat_aug)


# speedup vs baseline: 1.1221x; 1.0458x over previous
import jax
import jax.numpy as jnp
from jax.experimental import pallas as pl
from jax.experimental.pallas import tpu as pltpu


def _fused_body(a_ref, x1_ref, x2_ref, wz_ref, wx_ref,
                z1_ref, z2_ref, xh1_ref, xh2_ref, s2_ref):
    phase = pl.program_id(0)
    i = pl.program_id(1)
    f = wz_ref.shape[1]
    bm = a_ref.shape[0]

    @pl.when(phase == 0)
    def _layer1():
        t1 = jnp.dot(a_ref[...], x1_ref[...],
                     preferred_element_type=jnp.float32)
        t2 = jnp.dot(a_ref[...], x2_ref[...],
                     preferred_element_type=jnp.float32)
        z1 = jnp.dot(t1, wz_ref[...], preferred_element_type=jnp.float32)
        z2 = jnp.dot(t2, wz_ref[...], preferred_element_type=jnp.float32)
        z1_ref[...] = z1
        z2_ref[...] = z2
        s2_ref[pl.ds(i * bm, bm), :f] = jnp.dot(
            z1, wx_ref[...], preferred_element_type=jnp.float32)
        s2_ref[pl.ds(i * bm, bm), f:] = jnp.dot(
            z2, wx_ref[...], preferred_element_type=jnp.float32)

    @pl.when(phase == 1)
    def _layer2():
        out = jnp.dot(a_ref[...], s2_ref[...],
                      preferred_element_type=jnp.float32)
        xh1_ref[...] = out[:, :f]
        xh2_ref[...] = out[:, f:]


def _pick_block(n, target):
    for bm in range(min(target, n), 7, -1):
        if n % bm == 0 and bm % 8 == 0:
            return bm
    return n


def kernel(feat, feat_a, fadj, W_z, W_x):
    n, fin = feat.shape
    fhid = W_z.shape[1]
    fout = W_x.shape[1]
    bm = _pick_block(n, 400)
    nb = n // bm

    res = pl.pallas_call(
        _fused_body,
        grid=(2, nb),
        in_specs=[
            pl.BlockSpec((bm, n), lambda l, i: (i, 0)),
            pl.BlockSpec((n, fin), lambda l, i: (0, 0)),
            pl.BlockSpec((n, fin), lambda l, i: (0, 0)),
            pl.BlockSpec((fin, fhid), lambda l, i: (0, 0)),
            pl.BlockSpec((fhid, fout), lambda l, i: (0, 0)),
        ],
        out_specs=[
            pl.BlockSpec((bm, fhid), lambda l, i: ((1 - l) * i + l * (nb - 1), 0)),
            pl.BlockSpec((bm, fhid), lambda l, i: ((1 - l) * i + l * (nb - 1), 0)),
            pl.BlockSpec((bm, fout), lambda l, i: (l * i, 0)),
            pl.BlockSpec((bm, fout), lambda l, i: (l * i, 0)),
        ],
        out_shape=[
            jax.ShapeDtypeStruct((n, fhid), jnp.float32),
            jax.ShapeDtypeStruct((n, fhid), jnp.float32),
            jax.ShapeDtypeStruct((n, fout), jnp.float32),
            jax.ShapeDtypeStruct((n, fout), jnp.float32),
        ],
        scratch_shapes=[pltpu.VMEM((n, 2 * fout), jnp.float32)],
        compiler_params=pltpu.CompilerParams(
            dimension_semantics=("arbitrary", "arbitrary")),
    )(fadj, feat, feat_a, W_z, W_x)

    z_ori, z_aug, xhat_ori, xhat_aug = res
    return (z_ori, z_aug, xhat_ori, xhat_aug)
